# Initial kernel scaffold; baseline (speedup 1.0000x reference)
import jax
import jax.numpy as jnp
from jax.experimental import pallas as pl


def kernel(x, edge_index, batch, W_rel, b_rel, W_root):
    raise NotImplementedError("write your pallas kernel here")


# speedup vs baseline: 5.5798x; 5.5798x over previous
import functools

import jax
import jax.numpy as jnp
from jax import lax
from jax.experimental import pallas as pl
from jax.experimental.pallas import tpu as pltpu
from jax.experimental.pallas import tpu_sc as plsc

N = 10000
E = 320000
D = 128
B = 64

NC, NS, L = 2, 16, 16
NW = NC * NS
NPAD = 10240
SPT = NPAD // NW
EC = 128
NCH = 80
EPT = E // NW
EPT2 = NCH * EC
HN = NPAD // 2
AGR = HN + EC
HSL = HN // NS
RB = 10
RBS = N // RB
RBS3 = NPAD // RB

_mesh = plsc.VectorSubcoreMesh(core_axis_name="c", subcore_axis_name="s")
_scp = pltpu.CompilerParams(needs_layout_passes=False)


def _k1_body(x_ref, b2_ref, wt_ref, t_ref, cnt_ref, st_ref):
    i = pl.program_id(0)
    xb16 = x_ref[...].astype(jnp.bfloat16).astype(jnp.float32)
    wt16 = wt_ref[...].astype(jnp.bfloat16).astype(jnp.float32)
    t_ref[...] = jnp.sum(xb16 * wt16, axis=1, keepdims=True)
    bb = b2_ref[...]
    gid = lax.broadcasted_iota(jnp.int32, (RBS, B), 1)
    pc = jnp.sum((bb == gid).astype(jnp.int32), axis=0, keepdims=True)
    ps = jnp.sum((bb < gid).astype(jnp.int32), axis=0, keepdims=True)
    pc8 = jnp.broadcast_to(pc, (8, B))
    ps8 = jnp.broadcast_to(ps, (8, B))

    @pl.when(i == 0)
    def _():
        cnt_ref[...] = pc8
        st_ref[...] = ps8

    @pl.when(i > 0)
    def _():
        cnt_ref[...] += pc8
        st_ref[...] += ps8


_k1 = pl.pallas_call(
    _k1_body,
    grid=(RB,),
    in_specs=[
        pl.BlockSpec((RBS, D), lambda i: (i, 0)),
        pl.BlockSpec((RBS, 1), lambda i: (i, 0)),
        pl.BlockSpec((1, D), lambda i: (0, 0)),
    ],
    out_specs=[
        pl.BlockSpec((RBS, 1), lambda i: (i, 0)),
        pl.BlockSpec((8, B), lambda i: (0, 0)),
        pl.BlockSpec((8, B), lambda i: (0, 0)),
    ],
    out_shape=[
        jax.ShapeDtypeStruct((N, 1), jnp.float32),
        jax.ShapeDtypeStruct((8, B), jnp.int32),
        jax.ShapeDtypeStruct((8, B), jnp.int32),
    ],
)


@functools.partial(
    pl.kernel,
    mesh=_mesh,
    compiler_params=_scp,
    out_type=(
        jax.ShapeDtypeStruct((NC * NW * EPT2,), jnp.int32),
        jax.ShapeDtypeStruct((NC * NW * EPT2,), jnp.int32),
        jax.ShapeDtypeStruct((NW * 16,), jnp.int32),
    ),
    scratch_types=[
        pltpu.VMEM((EPT,), jnp.int32),
        pltpu.VMEM((EPT,), jnp.int32),
        pltpu.VMEM((EPT2,), jnp.int32),
        pltpu.VMEM((EPT2,), jnp.int32),
        pltpu.VMEM((EPT2,), jnp.int32),
        pltpu.VMEM((EPT2,), jnp.int32),
        pltpu.VMEM((16,), jnp.int32),
    ],
)
def _k2p(src_hbm, dst_hbm, srcl_hbm, dstl_hbm, cnt_hbm,
         sb, db, las, lad, lbs, lbd, rowb):
    cid = lax.axis_index("c")
    sid = lax.axis_index("s")
    wid = sid * NC + cid
    pltpu.sync_copy(src_hbm.at[pl.ds(wid * EPT, EPT)], sb)
    pltpu.sync_copy(dst_hbm.at[pl.ds(wid * EPT, EPT)], db)
    lane = lax.iota(jnp.int32, L)

    def split(i, carry):
        offa, offb = carry
        s16 = sb[pl.ds(i * L, L)]
        d16 = db[pl.ds(i * L, L)]
        ma = d16 < HN
        na = plsc.all_reduce_population_count(ma)[0]
        plsc.store_compressed(las.at[pl.ds(offa, L)], s16, mask=ma)
        plsc.store_compressed(lad.at[pl.ds(offa, L)], d16, mask=ma)
        mb = jnp.logical_not(ma)
        plsc.store_compressed(lbs.at[pl.ds(offb, L)], s16, mask=mb)
        plsc.store_compressed(lbd.at[pl.ds(offb, L)], d16 - HN, mask=mb)
        return offa + na, offb + (L - na)

    offa, offb = lax.fori_loop(0, EPT // L, split,
                               (jnp.int32(0), jnp.int32(0)))

    sdump = jnp.zeros((L,), jnp.int32)
    ddump = jnp.full((L,), HN, jnp.int32)
    nchs = []
    for h, (ls_, ld_, off) in enumerate(((las, lad, offa), (lbs, lbd, offb))):
        target = ((off + EC - 1) // EC) * EC
        for m in range(EC // L):
            o = off + m * L

            @pl.when(o < target)
            def _(ls_=ls_, ld_=ld_, o=o):
                ls_[pl.ds(o, L)] = sdump
                ld_[pl.ds(o, L)] = ddump

        nchs.append(target // EC)
        base = (h * NW + wid) * EPT2
        pltpu.sync_copy(ls_, srcl_hbm.at[pl.ds(base, EPT2)])
        pltpu.sync_copy(ld_, dstl_hbm.at[pl.ds(base, EPT2)])

    row = jnp.where(lane == 0, nchs[0], jnp.where(lane == 1, nchs[1], 0))
    rowb[...] = row
    pltpu.sync_copy(rowb, cnt_hbm.at[pl.ds(wid * 16, 16)])


@functools.partial(
    pl.kernel,
    mesh=_mesh,
    compiler_params=_scp,
    out_type=jax.ShapeDtypeStruct((NPAD, D), jnp.float32),
    scratch_types=[
        pltpu.VMEM((NCH, EC), jnp.int32),
        pltpu.VMEM((NCH, EC), jnp.int32),
        pltpu.VMEM((NW * 16,), jnp.int32),
        pltpu.VMEM((EC, D), jnp.float32),
        pltpu.VMEM((EC, D), jnp.float32),
        pltpu.VMEM((64, D), jnp.float32),
        pltpu.VMEM_SHARED((AGR, D), jnp.float32),
        pltpu.SemaphoreType.DMA,
        pltpu.SemaphoreType.DMA,
        pltpu.SemaphoreType.DMA,
        pltpu.SemaphoreType.DMA,
    ],
)
def _k2m(x_hbm, srcl_hbm, dstl_hbm, cnt_hbm, apc_hbm,
         srcb, dstb, cntb, xb0, xb1, zb, aggsp, gs0, gs1, ss0, ss1):
    cid = lax.axis_index("c")
    sid = lax.axis_index("s")
    lane = lax.iota(jnp.int32, L)
    pltpu.sync_copy(cnt_hbm, cntb)

    zero16 = jnp.zeros((L,), jnp.float32)

    def zrow(i, _):
        r = i // (D // L)
        c = i % (D // L)
        zb[r, pl.ds(c * L, L)] = zero16
        return 0

    lax.fori_loop(0, 64 * (D // L), zrow, 0)

    def zout(m, _):
        pltpu.sync_copy(zb, aggsp.at[pl.ds(sid * HSL + m * 64, 64)])
        return 0

    lax.fori_loop(0, HSL // 64, zout, 0)
    plsc.subcore_barrier()

    xbufs = (xb0, xb1)
    gsems = (gs0, gs1)
    ssems = (ss0, ss1)

    for li in range(2):
        ls = sid + NS * li
        cr = cntb[pl.ds(ls * 16, L)]
        nch = jnp.sum(jnp.where(lane == cid, cr, 0))
        lrow = cid * NW + ls
        pltpu.sync_copy(srcl_hbm.at[lrow], srcb)
        pltpu.sync_copy(dstl_hbm.at[lrow], dstb)

        @pl.when(nch > 0)
        def _():
            pltpu.async_copy(x_hbm.at[srcb.at[0]], xb0, gs0)

        @pl.when(nch > 1)
        def _():
            pltpu.async_copy(x_hbm.at[srcb.at[1]], xb1, gs1)

        def ring(m, _):
            for b in range(2):
                c = m * 2 + b

                @pl.when(c < nch)
                def _(b=b, c=c):
                    pltpu.make_async_copy(x_hbm.at[srcb.at[c]], xbufs[b],
                                          gsems[b]).wait()
                    pltpu.async_copy(xbufs[b], aggsp.at[dstb.at[c]],
                                     ssems[b], add=True)
                    pltpu.make_async_copy(xbufs[b], aggsp.at[dstb.at[c]],
                                          ssems[b]).wait()

                    @pl.when(c + 2 < nch)
                    def _():
                        pltpu.async_copy(x_hbm.at[srcb.at[c + 2]], xbufs[b],
                                         gsems[b])
            return 0

        lax.fori_loop(0, (nch + 1) // 2, ring, 0)

    plsc.subcore_barrier()

    def wout(m, _):
        r = sid * HSL + m * 64
        pltpu.sync_copy(aggsp.at[pl.ds(r, 64)],
                        apc_hbm.at[pl.ds(cid * HN + r, 64)])
        return 0

    lax.fori_loop(0, HSL // 64, wout, 0)


def _score16(pre):
    e = jnp.exp(pre * 2.0)
    return 1.0 - 2.0 / (e + 1.0)


def _key16(sval):
    u = plsc.bitcast(sval, jnp.uint32)
    m = u >> jnp.uint32(31)
    msk = jnp.where(m == jnp.uint32(1),
                    jnp.uint32(0xFFFFFFFF), jnp.uint32(0x80000000))
    return u ^ msk


def _bf(v):
    u = plsc.bitcast(v, jnp.uint32)
    r = (u + jnp.uint32(0x7FFF) + ((u >> jnp.uint32(16)) & jnp.uint32(1)))
    return plsc.bitcast(r & jnp.uint32(0xFFFF0000), jnp.float32)


def _chunk_keys(b0, wr8, bscal, t16, lane):
    dots = jnp.zeros((L,), jnp.float32)
    for rr in range(L):
        acc = jnp.zeros((L,), jnp.float32)
        for cc in range(D // L):
            acc = acc + _bf(b0[rr, pl.ds(cc * L, L)]) * wr8[cc]
        dots = jnp.where(lane == rr, jnp.sum(acc), dots)
    pre = (dots + bscal) + t16
    sval = _score16(pre)
    return sval, _key16(sval)


@functools.partial(
    pl.kernel,
    mesh=_mesh,
    compiler_params=_scp,
    out_type=jax.ShapeDtypeStruct((B * 16,), jnp.int32),
    scratch_types=[
        pltpu.VMEM((B + L,), jnp.int32),
        pltpu.VMEM((B + L,), jnp.int32),
        pltpu.VMEM((NPAD,), jnp.uint32),
        pltpu.VMEM((L, D), jnp.float32),
        pltpu.VMEM((L,), jnp.float32),
        pltpu.VMEM((D,), jnp.float32),
        pltpu.VMEM((L,), jnp.float32),
        pltpu.VMEM((16,), jnp.int32),
    ],
)
def _k2b(apc_hbm, t_hbm, wr_hbm, br_hbm, cnt_hbm, st_hbm, thr_hbm,
         cntb, stb, keyb, b0, tb, wrb, brb, rowb):
    cid = lax.axis_index("c")
    sid = lax.axis_index("s")
    wid = sid * NC + cid
    pltpu.sync_copy(cnt_hbm.at[pl.ds(0, B)], cntb.at[pl.ds(0, B)])
    pltpu.sync_copy(st_hbm.at[pl.ds(0, B)], stb.at[pl.ds(0, B)])
    pltpu.sync_copy(wr_hbm, wrb)
    pltpu.sync_copy(br_hbm, brb)
    wr8 = [_bf(wrb[pl.ds(cc * L, L)]) for cc in range(D // L)]
    bscal = brb[...][0]
    lane = lax.iota(jnp.int32, L)

    for gi in range(B // NW):
        g = wid + NW * gi
        n = cntb[pl.ds(g, L)][0]
        s = stb[pl.ds(g, L)][0]
        k = (n + 1) // 2
        j0 = (s // L) * L
        jn = (s + n - j0 + (L - 1)) // L

        def chunk(m, _):
            row0 = j0 + m * L
            pltpu.sync_copy(apc_hbm.at[pl.ds(row0, L)], b0)
            pltpu.sync_copy(t_hbm.at[pl.ds(row0, L)], tb)
            _, key = _chunk_keys(b0, wr8, bscal, tb[...], lane)
            keyb[pl.ds(row0, L)] = key
            return 0

        lax.fori_loop(0, jn, chunk, 0)

        def count_keys(v, strict):
            def body(m, acc):
                j = j0 + m * L
                idx = j + lane
                valid = (idx >= s) & (idx < s + n)
                k16 = keyb[pl.ds(j, L)]
                cmp = (k16 > v) if strict else (k16 >= v)
                return acc + jnp.where(valid & cmp, 1, 0)

            accv = lax.fori_loop(0, jn, body, jnp.zeros((L,), jnp.int32))
            return jnp.sum(accv)

        def bit_step(bi, p):
            cand = p | (jnp.uint32(1) << (jnp.uint32(31) - bi.astype(jnp.uint32)))
            c = count_keys(cand, strict=False)
            return jnp.where(c >= k, cand, p)

        thr = lax.fori_loop(0, 32, bit_step, jnp.uint32(0))
        n_gt = count_keys(thr, strict=True)
        quota = k - n_gt

        def cut_body(m, carry):
            cnt, cut = carry
            j = j0 + m * L
            idx = j + lane
            valid = (idx >= s) & (idx < s + n)
            k16 = keyb[pl.ds(j, L)]
            tie = (valid & (k16 == thr)).astype(jnp.int32)
            incl = plsc.cumsum(tie) + cnt
            hit = (tie == 1) & (incl == quota)
            cand = jnp.where(hit, idx, jnp.int32(2**31 - 1))
            return cnt + jnp.sum(tie), jnp.minimum(cut, jnp.min(cand))

        _, idx_cut = lax.fori_loop(0, jn, cut_body,
                                   (jnp.int32(0), jnp.int32(2**31 - 1)))

        row = jnp.where(lane == 0,
                        plsc.bitcast(jnp.broadcast_to(thr, (L,)), jnp.int32),
                        jnp.where(lane == 1, idx_cut, 0))
        rowb[...] = row
        pltpu.sync_copy(rowb, thr_hbm.at[pl.ds(g * 16, 16)])


@functools.partial(
    pl.kernel,
    mesh=_mesh,
    compiler_params=_scp,
    out_type=jax.ShapeDtypeStruct((NPAD,), jnp.float32),
    scratch_types=[
        pltpu.VMEM((L, D), jnp.float32),
        pltpu.VMEM((SPT,), jnp.float32),
        pltpu.VMEM((SPT,), jnp.int32),
        pltpu.VMEM((B * 16,), jnp.int32),
        pltpu.VMEM((D,), jnp.float32),
        pltpu.VMEM((L,), jnp.float32),
        pltpu.VMEM((SPT,), jnp.float32),
    ],
)
def _k2c(apc_hbm, t_hbm, wr_hbm, br_hbm, batch_hbm, thr_hbm, w_hbm,
         b0, tb, btb, thrb, wrb, brb, wb):
    cid = lax.axis_index("c")
    sid = lax.axis_index("s")
    wid = sid * NC + cid
    off0 = wid * SPT
    pltpu.sync_copy(t_hbm.at[pl.ds(off0, SPT)], tb)
    pltpu.sync_copy(batch_hbm.at[pl.ds(off0, SPT)], btb)
    pltpu.sync_copy(thr_hbm, thrb)
    pltpu.sync_copy(wr_hbm, wrb)
    pltpu.sync_copy(br_hbm, brb)
    wr8 = [_bf(wrb[pl.ds(cc * L, L)]) for cc in range(D // L)]
    bscal = brb[...][0]
    lane = lax.iota(jnp.int32, L)

    def body(j, _):
        row0 = off0 + j * L
        pltpu.sync_copy(apc_hbm.at[pl.ds(row0, L)], b0)
        sval, key = _chunk_keys(b0, wr8, bscal, tb[pl.ds(j * L, L)], lane)
        g16 = btb[pl.ds(j * L, L)] * 16
        thr_u = plsc.bitcast(plsc.load_gather(thrb, [g16]), jnp.uint32)
        cut_i = plsc.load_gather(thrb, [g16 + 1])
        idx = row0 + lane
        keep = (key > thr_u) | ((key == thr_u) & (idx <= cut_i))
        wb[pl.ds(j * L, L)] = jnp.where(keep, sval, 0.0)
        return 0

    lax.fori_loop(0, SPT // L, body, 0)
    pltpu.sync_copy(wb, w_hbm.at[pl.ds(off0, SPT)])


def _k3_body(x_ref, br_ref, wr_ref, cnt_ref, out_ref):
    i = pl.program_id(0)
    xb = x_ref[...]
    bt = br_ref[...]
    wv = wr_ref[...]
    gid = lax.broadcasted_iota(jnp.int32, (B, RBS3), 0)
    Wm = jnp.where(bt == gid, wv, 0.0)
    part = jnp.dot(Wm, xb, preferred_element_type=jnp.float32)

    @pl.when(i == 0)
    def _():
        out_ref[...] = part

    @pl.when(i > 0)
    def _():
        out_ref[...] += part

    @pl.when(i == RB - 1)
    def _():
        c = cnt_ref[...][0:1, :]
        kk = (c + 1) // 2
        winv = 1.0 / jnp.maximum(kk, 1).astype(jnp.float32)
        gr = lax.broadcasted_iota(jnp.int32, (B, B), 0)
        gc = lax.broadcasted_iota(jnp.int32, (B, B), 1)
        Dm = jnp.where(gr == gc, jnp.broadcast_to(winv, (B, B)), 0.0)
        out_ref[...] = jnp.dot(Dm, out_ref[...],
                               preferred_element_type=jnp.float32)


_k3 = pl.pallas_call(
    _k3_body,
    grid=(RB,),
    in_specs=[
        pl.BlockSpec((RBS3, D), lambda i: (i, 0)),
        pl.BlockSpec((1, RBS3), lambda i: (0, i)),
        pl.BlockSpec((1, RBS3), lambda i: (0, i)),
        pl.BlockSpec((8, B), lambda i: (0, 0)),
    ],
    out_specs=pl.BlockSpec((B, D), lambda i: (0, 0)),
    out_shape=jax.ShapeDtypeStruct((B, D), jnp.float32),
)


def kernel(x, edge_index, batch, W_rel, b_rel, W_root):
    batch = batch.astype(jnp.int32)
    src = edge_index[0].astype(jnp.int32)
    dst = edge_index[1].astype(jnp.int32)
    t, cnts, strts = _k1(x, batch.reshape(N, 1), W_root.reshape(1, D))
    tpad = jnp.pad(t.reshape(N), (0, NPAD - N))
    srcl, dstl, ecnts = _k2p(src, dst)
    apc = _k2m(x, srcl.reshape(NC * NW, NCH, EC),
               dstl.reshape(NC * NW, NCH, EC), ecnts)
    wr_flat = W_rel.reshape(D)
    br16 = jnp.pad(b_rel, (0, L - 1))
    thrpack = _k2b(apc, tpad, wr_flat, br16, cnts.reshape(-1),
                   strts.reshape(-1))
    w = _k2c(apc, tpad, wr_flat, br16, jnp.pad(batch, (0, NPAD - N)), thrpack)
    xpad = jnp.pad(x, ((0, NPAD - N), (0, 0)))
    bpad = jnp.pad(batch, (0, NPAD - N), constant_values=B)
    return _k3(xpad, bpad.reshape(1, NPAD), w.reshape(1, NPAD), cnts)


# speedup vs baseline: 5.6581x; 1.0140x over previous
import functools

import jax
import jax.numpy as jnp
from jax import lax
from jax.experimental import pallas as pl
from jax.experimental.pallas import tpu as pltpu
from jax.experimental.pallas import tpu_sc as plsc

N = 10000
E = 320000
D = 128
B = 64

NC, NS, L = 2, 16, 16
NW = NC * NS
NPAD = 10240
SPT = NPAD // NW
EC = 128
NCH = 80
EPT = E // NW
EPT2 = NCH * EC
HN = NPAD // 2
AGR = HN + EC
HSL = HN // NS
RB = 10
RBS = N // RB
RBS3 = NPAD // RB

_mesh = plsc.VectorSubcoreMesh(core_axis_name="c", subcore_axis_name="s")
_scp = pltpu.CompilerParams(needs_layout_passes=False)


def _k1_body(x_ref, b2_ref, wt_ref, t_ref, cnt_ref, st_ref):
    i = pl.program_id(0)
    xb16 = x_ref[...].astype(jnp.bfloat16).astype(jnp.float32)
    wt16 = wt_ref[...].astype(jnp.bfloat16).astype(jnp.float32)
    t_ref[...] = jnp.sum(xb16 * wt16, axis=1, keepdims=True)
    bb = b2_ref[...]
    gid = lax.broadcasted_iota(jnp.int32, (RBS, B), 1)
    pc = jnp.sum((bb == gid).astype(jnp.int32), axis=0, keepdims=True)
    ps = jnp.sum((bb < gid).astype(jnp.int32), axis=0, keepdims=True)
    pc8 = jnp.broadcast_to(pc, (8, B))
    ps8 = jnp.broadcast_to(ps, (8, B))

    @pl.when(i == 0)
    def _():
        cnt_ref[...] = pc8
        st_ref[...] = ps8

    @pl.when(i > 0)
    def _():
        cnt_ref[...] += pc8
        st_ref[...] += ps8


_k1 = pl.pallas_call(
    _k1_body,
    grid=(RB,),
    in_specs=[
        pl.BlockSpec((RBS, D), lambda i: (i, 0)),
        pl.BlockSpec((RBS, 1), lambda i: (i, 0)),
        pl.BlockSpec((1, D), lambda i: (0, 0)),
    ],
    out_specs=[
        pl.BlockSpec((RBS, 1), lambda i: (i, 0)),
        pl.BlockSpec((8, B), lambda i: (0, 0)),
        pl.BlockSpec((8, B), lambda i: (0, 0)),
    ],
    out_shape=[
        jax.ShapeDtypeStruct((N, 1), jnp.float32),
        jax.ShapeDtypeStruct((8, B), jnp.int32),
        jax.ShapeDtypeStruct((8, B), jnp.int32),
    ],
)


@functools.partial(
    pl.kernel,
    mesh=_mesh,
    compiler_params=_scp,
    out_type=(
        jax.ShapeDtypeStruct((NC * NW * EPT2,), jnp.int32),
        jax.ShapeDtypeStruct((NC * NW * EPT2,), jnp.int32),
        jax.ShapeDtypeStruct((NW * 16,), jnp.int32),
    ),
    scratch_types=[
        pltpu.VMEM((EPT,), jnp.int32),
        pltpu.VMEM((EPT,), jnp.int32),
        pltpu.VMEM((EPT2,), jnp.int32),
        pltpu.VMEM((EPT2,), jnp.int32),
        pltpu.VMEM((EPT2,), jnp.int32),
        pltpu.VMEM((EPT2,), jnp.int32),
        pltpu.VMEM((16,), jnp.int32),
    ],
)
def _k2p(src_hbm, dst_hbm, srcl_hbm, dstl_hbm, cnt_hbm,
         sb, db, las, lad, lbs, lbd, rowb):
    cid = lax.axis_index("c")
    sid = lax.axis_index("s")
    wid = sid * NC + cid
    pltpu.sync_copy(src_hbm.at[pl.ds(wid * EPT, EPT)], sb)
    pltpu.sync_copy(dst_hbm.at[pl.ds(wid * EPT, EPT)], db)
    lane = lax.iota(jnp.int32, L)

    def split(i, carry):
        offa, offb = carry
        s16 = sb[pl.ds(i * L, L)]
        d16 = db[pl.ds(i * L, L)]
        ma = d16 < HN
        na = plsc.all_reduce_population_count(ma)[0]
        plsc.store_compressed(las.at[pl.ds(offa, L)], s16, mask=ma)
        plsc.store_compressed(lad.at[pl.ds(offa, L)], d16, mask=ma)
        mb = jnp.logical_not(ma)
        plsc.store_compressed(lbs.at[pl.ds(offb, L)], s16, mask=mb)
        plsc.store_compressed(lbd.at[pl.ds(offb, L)], d16 - HN, mask=mb)
        return offa + na, offb + (L - na)

    offa, offb = lax.fori_loop(0, EPT // L, split,
                               (jnp.int32(0), jnp.int32(0)))

    sdump = jnp.zeros((L,), jnp.int32)
    ddump = jnp.full((L,), HN, jnp.int32)
    nchs = []
    for h, (ls_, ld_, off) in enumerate(((las, lad, offa), (lbs, lbd, offb))):
        target = ((off + EC - 1) // EC) * EC
        for m in range(EC // L):
            o = off + m * L

            @pl.when(o < target)
            def _(ls_=ls_, ld_=ld_, o=o):
                ls_[pl.ds(o, L)] = sdump
                ld_[pl.ds(o, L)] = ddump

        nchs.append(target // EC)
        base = (h * NW + wid) * EPT2
        pltpu.sync_copy(ls_, srcl_hbm.at[pl.ds(base, EPT2)])
        pltpu.sync_copy(ld_, dstl_hbm.at[pl.ds(base, EPT2)])

    row = jnp.where(lane == 0, nchs[0], jnp.where(lane == 1, nchs[1], 0))
    rowb[...] = row
    pltpu.sync_copy(rowb, cnt_hbm.at[pl.ds(wid * 16, 16)])


@functools.partial(
    pl.kernel,
    mesh=_mesh,
    compiler_params=_scp,
    out_type=jax.ShapeDtypeStruct((NPAD, D), jnp.float32),
    scratch_types=[
        pltpu.VMEM((NCH, EC), jnp.int32),
        pltpu.VMEM((NCH, EC), jnp.int32),
        pltpu.VMEM((NW * 16,), jnp.int32),
        pltpu.VMEM((EC, D), jnp.float32),
        pltpu.VMEM((EC, D), jnp.float32),
        pltpu.VMEM((EC, D), jnp.float32),
        pltpu.VMEM((64, D), jnp.float32),
        pltpu.VMEM_SHARED((AGR, D), jnp.float32),
        pltpu.SemaphoreType.DMA,
        pltpu.SemaphoreType.DMA,
        pltpu.SemaphoreType.DMA,
        pltpu.SemaphoreType.DMA,
        pltpu.SemaphoreType.DMA,
        pltpu.SemaphoreType.DMA,
    ],
)
def _k2m(x_hbm, srcl_hbm, dstl_hbm, cnt_hbm, apc_hbm,
         srcb, dstb, cntb, xb0, xb1, xb2, zb, aggsp,
         gs0, gs1, gs2, ss0, ss1, ss2):
    cid = lax.axis_index("c")
    sid = lax.axis_index("s")
    lane = lax.iota(jnp.int32, L)
    pltpu.sync_copy(cnt_hbm, cntb)

    zero16 = jnp.zeros((L,), jnp.float32)

    def zrow(i, _):
        r = i // (D // L)
        c = i % (D // L)
        zb[r, pl.ds(c * L, L)] = zero16
        return 0

    lax.fori_loop(0, 64 * (D // L), zrow, 0)

    def zout(m, _):
        pltpu.sync_copy(zb, aggsp.at[pl.ds(sid * HSL + m * 64, 64)])
        return 0

    lax.fori_loop(0, HSL // 64, zout, 0)
    plsc.subcore_barrier()

    xbufs = (xb0, xb1, xb2)
    gsems = (gs0, gs1, gs2)
    ssems = (ss0, ss1, ss2)
    NBUF = 3

    for li in range(2):
        ls = sid + NS * li
        cr = cntb[pl.ds(ls * 16, L)]
        nch = jnp.sum(jnp.where(lane == cid, cr, 0))
        lrow = cid * NW + ls
        pltpu.sync_copy(srcl_hbm.at[lrow], srcb)
        pltpu.sync_copy(dstl_hbm.at[lrow], dstb)

        for p in range(NBUF):
            @pl.when(nch > p)
            def _(p=p):
                pltpu.async_copy(x_hbm.at[srcb.at[p]], xbufs[p], gsems[p])

        def ring(m, _):
            for b in range(NBUF):
                c = m * NBUF + b

                @pl.when(c < nch)
                def _(b=b, c=c):
                    pltpu.make_async_copy(x_hbm.at[srcb.at[c]], xbufs[b],
                                          gsems[b]).wait()
                    pltpu.async_copy(xbufs[b], aggsp.at[dstb.at[c]],
                                     ssems[b], add=True)
                    pltpu.make_async_copy(xbufs[b], aggsp.at[dstb.at[c]],
                                          ssems[b]).wait()

                    @pl.when(c + NBUF < nch)
                    def _():
                        pltpu.async_copy(x_hbm.at[srcb.at[c + NBUF]],
                                         xbufs[b], gsems[b])
            return 0

        lax.fori_loop(0, (nch + NBUF - 1) // NBUF, ring, 0)

    plsc.subcore_barrier()

    def wout(m, _):
        r = sid * HSL + m * 64
        pltpu.sync_copy(aggsp.at[pl.ds(r, 64)],
                        apc_hbm.at[pl.ds(cid * HN + r, 64)])
        return 0

    lax.fori_loop(0, HSL // 64, wout, 0)


def _score16(pre):
    e = jnp.exp(pre * 2.0)
    return 1.0 - 2.0 / (e + 1.0)


def _key16(sval):
    u = plsc.bitcast(sval, jnp.uint32)
    m = u >> jnp.uint32(31)
    msk = jnp.where(m == jnp.uint32(1),
                    jnp.uint32(0xFFFFFFFF), jnp.uint32(0x80000000))
    return u ^ msk


def _bf(v):
    u = plsc.bitcast(v, jnp.uint32)
    r = (u + jnp.uint32(0x7FFF) + ((u >> jnp.uint32(16)) & jnp.uint32(1)))
    return plsc.bitcast(r & jnp.uint32(0xFFFF0000), jnp.float32)


def _chunk_keys(b0, wr8, bscal, t16, lane):
    dots = jnp.zeros((L,), jnp.float32)
    for rr in range(L):
        acc = jnp.zeros((L,), jnp.float32)
        for cc in range(D // L):
            acc = acc + _bf(b0[rr, pl.ds(cc * L, L)]) * wr8[cc]
        dots = jnp.where(lane == rr, jnp.sum(acc), dots)
    pre = (dots + bscal) + t16
    sval = _score16(pre)
    return sval, _key16(sval)


@functools.partial(
    pl.kernel,
    mesh=_mesh,
    compiler_params=_scp,
    out_type=jax.ShapeDtypeStruct((B * 16,), jnp.int32),
    scratch_types=[
        pltpu.VMEM((B + L,), jnp.int32),
        pltpu.VMEM((B + L,), jnp.int32),
        pltpu.VMEM((NPAD,), jnp.uint32),
        pltpu.VMEM((L, D), jnp.float32),
        pltpu.VMEM((L,), jnp.float32),
        pltpu.VMEM((D,), jnp.float32),
        pltpu.VMEM((L,), jnp.float32),
        pltpu.VMEM((16,), jnp.int32),
    ],
)
def _k2b(apc_hbm, t_hbm, wr_hbm, br_hbm, cnt_hbm, st_hbm, thr_hbm,
         cntb, stb, keyb, b0, tb, wrb, brb, rowb):
    cid = lax.axis_index("c")
    sid = lax.axis_index("s")
    wid = sid * NC + cid
    pltpu.sync_copy(cnt_hbm.at[pl.ds(0, B)], cntb.at[pl.ds(0, B)])
    pltpu.sync_copy(st_hbm.at[pl.ds(0, B)], stb.at[pl.ds(0, B)])
    pltpu.sync_copy(wr_hbm, wrb)
    pltpu.sync_copy(br_hbm, brb)
    wr8 = [_bf(wrb[pl.ds(cc * L, L)]) for cc in range(D // L)]
    bscal = brb[...][0]
    lane = lax.iota(jnp.int32, L)

    for gi in range(B // NW):
        g = wid + NW * gi
        n = cntb[pl.ds(g, L)][0]
        s = stb[pl.ds(g, L)][0]
        k = (n + 1) // 2
        j0 = (s // L) * L
        jn = (s + n - j0 + (L - 1)) // L

        def chunk(m, _):
            row0 = j0 + m * L
            pltpu.sync_copy(apc_hbm.at[pl.ds(row0, L)], b0)
            pltpu.sync_copy(t_hbm.at[pl.ds(row0, L)], tb)
            _, key = _chunk_keys(b0, wr8, bscal, tb[...], lane)
            keyb[pl.ds(row0, L)] = key
            return 0

        lax.fori_loop(0, jn, chunk, 0)

        def count_keys(v, strict):
            def body(m, acc):
                j = j0 + m * L
                idx = j + lane
                valid = (idx >= s) & (idx < s + n)
                k16 = keyb[pl.ds(j, L)]
                cmp = (k16 > v) if strict else (k16 >= v)
                return acc + jnp.where(valid & cmp, 1, 0)

            accv = lax.fori_loop(0, jn, body, jnp.zeros((L,), jnp.int32))
            return jnp.sum(accv)

        def bit_step(bi, p):
            cand = p | (jnp.uint32(1) << (jnp.uint32(31) - bi.astype(jnp.uint32)))
            c = count_keys(cand, strict=False)
            return jnp.where(c >= k, cand, p)

        thr = lax.fori_loop(0, 32, bit_step, jnp.uint32(0))
        n_gt = count_keys(thr, strict=True)
        quota = k - n_gt

        def cut_body(m, carry):
            cnt, cut = carry
            j = j0 + m * L
            idx = j + lane
            valid = (idx >= s) & (idx < s + n)
            k16 = keyb[pl.ds(j, L)]
            tie = (valid & (k16 == thr)).astype(jnp.int32)
            incl = plsc.cumsum(tie) + cnt
            hit = (tie == 1) & (incl == quota)
            cand = jnp.where(hit, idx, jnp.int32(2**31 - 1))
            return cnt + jnp.sum(tie), jnp.minimum(cut, jnp.min(cand))

        _, idx_cut = lax.fori_loop(0, jn, cut_body,
                                   (jnp.int32(0), jnp.int32(2**31 - 1)))

        row = jnp.where(lane == 0,
                        plsc.bitcast(jnp.broadcast_to(thr, (L,)), jnp.int32),
                        jnp.where(lane == 1, idx_cut, 0))
        rowb[...] = row
        pltpu.sync_copy(rowb, thr_hbm.at[pl.ds(g * 16, 16)])


@functools.partial(
    pl.kernel,
    mesh=_mesh,
    compiler_params=_scp,
    out_type=jax.ShapeDtypeStruct((NPAD,), jnp.float32),
    scratch_types=[
        pltpu.VMEM((L, D), jnp.float32),
        pltpu.VMEM((SPT,), jnp.float32),
        pltpu.VMEM((SPT,), jnp.int32),
        pltpu.VMEM((B * 16,), jnp.int32),
        pltpu.VMEM((D,), jnp.float32),
        pltpu.VMEM((L,), jnp.float32),
        pltpu.VMEM((SPT,), jnp.float32),
    ],
)
def _k2c(apc_hbm, t_hbm, wr_hbm, br_hbm, batch_hbm, thr_hbm, w_hbm,
         b0, tb, btb, thrb, wrb, brb, wb):
    cid = lax.axis_index("c")
    sid = lax.axis_index("s")
    wid = sid * NC + cid
    off0 = wid * SPT
    pltpu.sync_copy(t_hbm.at[pl.ds(off0, SPT)], tb)
    pltpu.sync_copy(batch_hbm.at[pl.ds(off0, SPT)], btb)
    pltpu.sync_copy(thr_hbm, thrb)
    pltpu.sync_copy(wr_hbm, wrb)
    pltpu.sync_copy(br_hbm, brb)
    wr8 = [_bf(wrb[pl.ds(cc * L, L)]) for cc in range(D // L)]
    bscal = brb[...][0]
    lane = lax.iota(jnp.int32, L)

    def body(j, _):
        row0 = off0 + j * L
        pltpu.sync_copy(apc_hbm.at[pl.ds(row0, L)], b0)
        sval, key = _chunk_keys(b0, wr8, bscal, tb[pl.ds(j * L, L)], lane)
        g16 = btb[pl.ds(j * L, L)] * 16
        thr_u = plsc.bitcast(plsc.load_gather(thrb, [g16]), jnp.uint32)
        cut_i = plsc.load_gather(thrb, [g16 + 1])
        idx = row0 + lane
        keep = (key > thr_u) | ((key == thr_u) & (idx <= cut_i))
        wb[pl.ds(j * L, L)] = jnp.where(keep, sval, 0.0)
        return 0

    lax.fori_loop(0, SPT // L, body, 0)
    pltpu.sync_copy(wb, w_hbm.at[pl.ds(off0, SPT)])


def _k3_body(x_ref, br_ref, wr_ref, cnt_ref, out_ref):
    i = pl.program_id(0)
    xb = x_ref[...]
    bt = br_ref[...]
    wv = wr_ref[...]
    gid = lax.broadcasted_iota(jnp.int32, (B, RBS3), 0)
    Wm = jnp.where(bt == gid, wv, 0.0)
    part = jnp.dot(Wm, xb, preferred_element_type=jnp.float32)

    @pl.when(i == 0)
    def _():
        out_ref[...] = part

    @pl.when(i > 0)
    def _():
        out_ref[...] += part

    @pl.when(i == RB - 1)
    def _():
        c = cnt_ref[...][0:1, :]
        kk = (c + 1) // 2
        winv = 1.0 / jnp.maximum(kk, 1).astype(jnp.float32)
        gr = lax.broadcasted_iota(jnp.int32, (B, B), 0)
        gc = lax.broadcasted_iota(jnp.int32, (B, B), 1)
        Dm = jnp.where(gr == gc, jnp.broadcast_to(winv, (B, B)), 0.0)
        out_ref[...] = jnp.dot(Dm, out_ref[...],
                               preferred_element_type=jnp.float32)


_k3 = pl.pallas_call(
    _k3_body,
    grid=(RB,),
    in_specs=[
        pl.BlockSpec((RBS3, D), lambda i: (i, 0)),
        pl.BlockSpec((1, RBS3), lambda i: (0, i)),
        pl.BlockSpec((1, RBS3), lambda i: (0, i)),
        pl.BlockSpec((8, B), lambda i: (0, 0)),
    ],
    out_specs=pl.BlockSpec((B, D), lambda i: (0, 0)),
    out_shape=jax.ShapeDtypeStruct((B, D), jnp.float32),
)


def kernel(x, edge_index, batch, W_rel, b_rel, W_root):
    batch = batch.astype(jnp.int32)
    src = edge_index[0].astype(jnp.int32)
    dst = edge_index[1].astype(jnp.int32)
    t, cnts, strts = _k1(x, batch.reshape(N, 1), W_root.reshape(1, D))
    tpad = jnp.pad(t.reshape(N), (0, NPAD - N))
    srcl, dstl, ecnts = _k2p(src, dst)
    apc = _k2m(x, srcl.reshape(NC * NW, NCH, EC),
               dstl.reshape(NC * NW, NCH, EC), ecnts)
    wr_flat = W_rel.reshape(D)
    br16 = jnp.pad(b_rel, (0, L - 1))
    thrpack = _k2b(apc, tpad, wr_flat, br16, cnts.reshape(-1),
                   strts.reshape(-1))
    w = _k2c(apc, tpad, wr_flat, br16, jnp.pad(batch, (0, NPAD - N)), thrpack)
    xpad = jnp.pad(x, ((0, NPAD - N), (0, 0)))
    bpad = jnp.pad(batch, (0, NPAD - N), constant_values=B)
    return _k3(xpad, bpad.reshape(1, NPAD), w.reshape(1, NPAD), cnts)


# speedup vs baseline: 6.1211x; 1.0818x over previous
import functools

import jax
import jax.numpy as jnp
from jax import lax
from jax.experimental import pallas as pl
from jax.experimental.pallas import tpu as pltpu
from jax.experimental.pallas import tpu_sc as plsc

N = 10000
E = 320000
D = 128
B = 64

NC, NS, L = 2, 16, 16
NW = NC * NS
NPAD = 10240
SPT = NPAD // NW
EC = 128
NCH = 80
EPT = E // NW
EPT2 = NCH * EC
HN = NPAD // 2
AGR = HN + EC
HSL = HN // NS
RB = 10
RBS = N // RB
RBS3 = NPAD // RB

_mesh = plsc.VectorSubcoreMesh(core_axis_name="c", subcore_axis_name="s")
_scp = pltpu.CompilerParams(needs_layout_passes=False)


def _k1_body(x_ref, b2_ref, wt_ref, t_ref, cnt_ref, st_ref):
    i = pl.program_id(0)
    xb16 = x_ref[...].astype(jnp.bfloat16).astype(jnp.float32)
    wt16 = wt_ref[...].astype(jnp.bfloat16).astype(jnp.float32)
    t_ref[...] = jnp.sum(xb16 * wt16, axis=1, keepdims=True)
    bb = b2_ref[...]
    gid = lax.broadcasted_iota(jnp.int32, (RBS, B), 1)
    pc = jnp.sum((bb == gid).astype(jnp.int32), axis=0, keepdims=True)
    ps = jnp.sum((bb < gid).astype(jnp.int32), axis=0, keepdims=True)
    pc8 = jnp.broadcast_to(pc, (8, B))
    ps8 = jnp.broadcast_to(ps, (8, B))

    @pl.when(i == 0)
    def _():
        cnt_ref[...] = pc8
        st_ref[...] = ps8

    @pl.when(i > 0)
    def _():
        cnt_ref[...] += pc8
        st_ref[...] += ps8


_k1 = pl.pallas_call(
    _k1_body,
    grid=(RB,),
    in_specs=[
        pl.BlockSpec((RBS, D), lambda i: (i, 0)),
        pl.BlockSpec((RBS, 1), lambda i: (i, 0)),
        pl.BlockSpec((1, D), lambda i: (0, 0)),
    ],
    out_specs=[
        pl.BlockSpec((RBS, 1), lambda i: (i, 0)),
        pl.BlockSpec((8, B), lambda i: (0, 0)),
        pl.BlockSpec((8, B), lambda i: (0, 0)),
    ],
    out_shape=[
        jax.ShapeDtypeStruct((N, 1), jnp.float32),
        jax.ShapeDtypeStruct((8, B), jnp.int32),
        jax.ShapeDtypeStruct((8, B), jnp.int32),
    ],
)


@functools.partial(
    pl.kernel,
    mesh=_mesh,
    compiler_params=_scp,
    out_type=(
        jax.ShapeDtypeStruct((NC * NW * EPT2,), jnp.int32),
        jax.ShapeDtypeStruct((NC * NW * EPT2,), jnp.int32),
        jax.ShapeDtypeStruct((NW * 16,), jnp.int32),
    ),
    scratch_types=[
        pltpu.VMEM((EPT,), jnp.int32),
        pltpu.VMEM((EPT,), jnp.int32),
        pltpu.VMEM((EPT2,), jnp.int32),
        pltpu.VMEM((EPT2,), jnp.int32),
        pltpu.VMEM((EPT2,), jnp.int32),
        pltpu.VMEM((EPT2,), jnp.int32),
        pltpu.VMEM((16,), jnp.int32),
    ],
)
def _k2p(src_hbm, dst_hbm, srcl_hbm, dstl_hbm, cnt_hbm,
         sb, db, las, lad, lbs, lbd, rowb):
    cid = lax.axis_index("c")
    sid = lax.axis_index("s")
    wid = sid * NC + cid
    pltpu.sync_copy(src_hbm.at[pl.ds(wid * EPT, EPT)], sb)
    pltpu.sync_copy(dst_hbm.at[pl.ds(wid * EPT, EPT)], db)
    lane = lax.iota(jnp.int32, L)

    def split(i, carry):
        offa, offb = carry
        s16 = sb[pl.ds(i * L, L)]
        d16 = db[pl.ds(i * L, L)]
        ma = d16 < HN
        na = plsc.all_reduce_population_count(ma)[0]
        plsc.store_compressed(las.at[pl.ds(offa, L)], s16, mask=ma)
        plsc.store_compressed(lad.at[pl.ds(offa, L)], d16, mask=ma)
        mb = jnp.logical_not(ma)
        plsc.store_compressed(lbs.at[pl.ds(offb, L)], s16, mask=mb)
        plsc.store_compressed(lbd.at[pl.ds(offb, L)], d16 - HN, mask=mb)
        return offa + na, offb + (L - na)

    offa, offb = lax.fori_loop(0, EPT // L, split,
                               (jnp.int32(0), jnp.int32(0)))

    sdump = jnp.zeros((L,), jnp.int32)
    ddump = jnp.full((L,), HN, jnp.int32)
    nchs = []
    for h, (ls_, ld_, off) in enumerate(((las, lad, offa), (lbs, lbd, offb))):
        target = ((off + EC - 1) // EC) * EC
        for m in range(EC // L):
            o = off + m * L

            @pl.when(o < target)
            def _(ls_=ls_, ld_=ld_, o=o):
                ls_[pl.ds(o, L)] = sdump
                ld_[pl.ds(o, L)] = ddump

        nchs.append(target // EC)
        base = (h * NW + wid) * EPT2
        pltpu.sync_copy(ls_, srcl_hbm.at[pl.ds(base, EPT2)])
        pltpu.sync_copy(ld_, dstl_hbm.at[pl.ds(base, EPT2)])

    row = jnp.where(lane == 0, nchs[0], jnp.where(lane == 1, nchs[1], 0))
    rowb[...] = row
    pltpu.sync_copy(rowb, cnt_hbm.at[pl.ds(wid * 16, 16)])


@functools.partial(
    pl.kernel,
    mesh=_mesh,
    compiler_params=_scp,
    out_type=jax.ShapeDtypeStruct((NPAD, D), jnp.float32),
    scratch_types=[
        pltpu.VMEM((NCH, EC), jnp.int32),
        pltpu.VMEM((NCH, EC), jnp.int32),
        pltpu.VMEM((NW * 16,), jnp.int32),
        pltpu.VMEM((EC, D), jnp.float32),
        pltpu.VMEM((EC, D), jnp.float32),
        pltpu.VMEM((EC, D), jnp.float32),
        pltpu.VMEM((64, D), jnp.float32),
        pltpu.VMEM_SHARED((AGR, D), jnp.float32),
        pltpu.SemaphoreType.DMA,
        pltpu.SemaphoreType.DMA,
        pltpu.SemaphoreType.DMA,
        pltpu.SemaphoreType.DMA,
        pltpu.SemaphoreType.DMA,
        pltpu.SemaphoreType.DMA,
    ],
)
def _k2m(x_hbm, srcl_hbm, dstl_hbm, cnt_hbm, apc_hbm,
         srcb, dstb, cntb, xb0, xb1, xb2, zb, aggsp,
         gs0, gs1, gs2, ss0, ss1, ss2):
    cid = lax.axis_index("c")
    sid = lax.axis_index("s")
    lane = lax.iota(jnp.int32, L)
    pltpu.sync_copy(cnt_hbm, cntb)

    zero16 = jnp.zeros((L,), jnp.float32)

    def zrow(i, _):
        r = i // (D // L)
        c = i % (D // L)
        zb[r, pl.ds(c * L, L)] = zero16
        return 0

    lax.fori_loop(0, 64 * (D // L), zrow, 0)

    def zout(m, _):
        pltpu.sync_copy(zb, aggsp.at[pl.ds(sid * HSL + m * 64, 64)])
        return 0

    lax.fori_loop(0, HSL // 64, zout, 0)
    plsc.subcore_barrier()

    xbufs = (xb0, xb1, xb2)
    gsems = (gs0, gs1, gs2)
    ssems = (ss0, ss1, ss2)
    NBUF = 3

    for li in range(2):
        ls = sid + NS * li
        cr = cntb[pl.ds(ls * 16, L)]
        nch = jnp.sum(jnp.where(lane == cid, cr, 0))
        lrow = cid * NW + ls
        pltpu.sync_copy(srcl_hbm.at[lrow], srcb)
        pltpu.sync_copy(dstl_hbm.at[lrow], dstb)

        for p in range(NBUF):
            @pl.when(nch > p)
            def _(p=p):
                pltpu.async_copy(x_hbm.at[srcb.at[p]], xbufs[p], gsems[p])

        def ring(m, _):
            for b in range(NBUF):
                c = m * NBUF + b

                @pl.when(c < nch)
                def _(b=b, c=c):
                    pltpu.make_async_copy(x_hbm.at[srcb.at[c]], xbufs[b],
                                          gsems[b]).wait()
                    pltpu.async_copy(xbufs[b], aggsp.at[dstb.at[c]],
                                     ssems[b], add=True)
                    pltpu.make_async_copy(xbufs[b], aggsp.at[dstb.at[c]],
                                          ssems[b]).wait()

                    @pl.when(c + NBUF < nch)
                    def _():
                        pltpu.async_copy(x_hbm.at[srcb.at[c + NBUF]],
                                         xbufs[b], gsems[b])
            return 0

        lax.fori_loop(0, (nch + NBUF - 1) // NBUF, ring, 0)

    plsc.subcore_barrier()

    def wout(m, _):
        r = sid * HSL + m * 64
        pltpu.sync_copy(aggsp.at[pl.ds(r, 64)],
                        apc_hbm.at[pl.ds(cid * HN + r, 64)])
        return 0

    lax.fori_loop(0, HSL // 64, wout, 0)


def _score16(pre):
    e = jnp.exp(pre * 2.0)
    return 1.0 - 2.0 / (e + 1.0)


def _key16(sval):
    u = plsc.bitcast(sval, jnp.uint32)
    m = u >> jnp.uint32(31)
    msk = jnp.where(m == jnp.uint32(1),
                    jnp.uint32(0xFFFFFFFF), jnp.uint32(0x80000000))
    return u ^ msk


def _bf(v):
    u = plsc.bitcast(v, jnp.uint32)
    r = (u + jnp.uint32(0x7FFF) + ((u >> jnp.uint32(16)) & jnp.uint32(1)))
    return plsc.bitcast(r & jnp.uint32(0xFFFF0000), jnp.float32)


def _chunk_keys(b0, wr8, bscal, t16, lane, row_off=0):
    dots = jnp.zeros((L,), jnp.float32)
    for rr in range(L):
        acc = jnp.zeros((L,), jnp.float32)
        for cc in range(D // L):
            acc = acc + _bf(b0[row_off + rr, pl.ds(cc * L, L)]) * wr8[cc]
        dots = jnp.where(lane == rr, jnp.sum(acc), dots)
    pre = (dots + bscal) + t16
    sval = _score16(pre)
    return sval, _key16(sval)


@functools.partial(
    pl.kernel,
    mesh=_mesh,
    compiler_params=_scp,
    out_type=jax.ShapeDtypeStruct((B * 16,), jnp.int32),
    scratch_types=[
        pltpu.VMEM((B + L,), jnp.int32),
        pltpu.VMEM((B + L,), jnp.int32),
        pltpu.VMEM((NPAD,), jnp.uint32),
        pltpu.VMEM((384, D), jnp.float32),
        pltpu.VMEM((L, D), jnp.float32),
        pltpu.VMEM((384,), jnp.float32),
        pltpu.VMEM((L,), jnp.float32),
        pltpu.VMEM((D,), jnp.float32),
        pltpu.VMEM((L,), jnp.float32),
        pltpu.VMEM((16,), jnp.int32),
    ],
)
def _k2b(apc_hbm, t_hbm, wr_hbm, br_hbm, cnt_hbm, st_hbm, thr_hbm,
         cntb, stb, keyb, b0, b1, tbb, tb, wrb, brb, rowb):
    cid = lax.axis_index("c")
    sid = lax.axis_index("s")
    wid = sid * NC + cid
    pltpu.sync_copy(cnt_hbm.at[pl.ds(0, B)], cntb.at[pl.ds(0, B)])
    pltpu.sync_copy(st_hbm.at[pl.ds(0, B)], stb.at[pl.ds(0, B)])
    pltpu.sync_copy(wr_hbm, wrb)
    pltpu.sync_copy(br_hbm, brb)
    wr8 = [_bf(wrb[pl.ds(cc * L, L)]) for cc in range(D // L)]
    bscal = brb[...][0]
    lane = lax.iota(jnp.int32, L)

    for gi in range(B // NW):
        g = wid + NW * gi
        n = cntb[pl.ds(g, L)][0]
        s = stb[pl.ds(g, L)][0]
        k = (n + 1) // 2
        j0 = (s // L) * L
        jn = (s + n - j0 + (L - 1)) // L
        nrows = jn * L

        loaded = jnp.int32(0)
        for blk in range(3):
            cond = (nrows > blk * EC) & (j0 + (blk + 1) * EC <= NPAD)
            loaded = jnp.where(cond, jnp.int32((blk + 1) * EC), loaded)

            @pl.when(cond)
            def _(blk=blk):
                pltpu.sync_copy(apc_hbm.at[pl.ds(j0 + blk * EC, EC)],
                                b0.at[pl.ds(blk * EC, EC)])
                pltpu.sync_copy(t_hbm.at[pl.ds(j0 + blk * EC, EC)],
                                tbb.at[pl.ds(blk * EC, EC)])

        jfast = jnp.minimum(jn, loaded // L)

        def chunk(m, _):
            _, key = _chunk_keys(b0, wr8, bscal, tbb[pl.ds(m * L, L)], lane,
                                 row_off=m * L)
            keyb[pl.ds(j0 + m * L, L)] = key
            return 0

        lax.fori_loop(0, jfast, chunk, 0)

        def chunk_tail(m, _):
            row0 = j0 + m * L
            pltpu.sync_copy(apc_hbm.at[pl.ds(row0, L)], b1)
            pltpu.sync_copy(t_hbm.at[pl.ds(row0, L)], tb)
            _, key = _chunk_keys(b1, wr8, bscal, tb[...], lane)
            keyb[pl.ds(row0, L)] = key
            return 0

        lax.fori_loop(jfast, jn, chunk_tail, 0)

        def count_keys(v, strict):
            def body(m, acc):
                j = j0 + m * L
                idx = j + lane
                valid = (idx >= s) & (idx < s + n)
                k16 = keyb[pl.ds(j, L)]
                cmp = (k16 > v) if strict else (k16 >= v)
                return acc + jnp.where(valid & cmp, 1, 0)

            accv = lax.fori_loop(0, jn, body, jnp.zeros((L,), jnp.int32))
            return jnp.sum(accv)

        def bit_step(bi, p):
            cand = p | (jnp.uint32(1) << (jnp.uint32(31) - bi.astype(jnp.uint32)))
            c = count_keys(cand, strict=False)
            return jnp.where(c >= k, cand, p)

        thr = lax.fori_loop(0, 32, bit_step, jnp.uint32(0))
        n_gt = count_keys(thr, strict=True)
        quota = k - n_gt

        def cut_body(m, carry):
            cnt, cut = carry
            j = j0 + m * L
            idx = j + lane
            valid = (idx >= s) & (idx < s + n)
            k16 = keyb[pl.ds(j, L)]
            tie = (valid & (k16 == thr)).astype(jnp.int32)
            incl = plsc.cumsum(tie) + cnt
            hit = (tie == 1) & (incl == quota)
            cand = jnp.where(hit, idx, jnp.int32(2**31 - 1))
            return cnt + jnp.sum(tie), jnp.minimum(cut, jnp.min(cand))

        _, idx_cut = lax.fori_loop(0, jn, cut_body,
                                   (jnp.int32(0), jnp.int32(2**31 - 1)))

        row = jnp.where(lane == 0,
                        plsc.bitcast(jnp.broadcast_to(thr, (L,)), jnp.int32),
                        jnp.where(lane == 1, idx_cut, 0))
        rowb[...] = row
        pltpu.sync_copy(rowb, thr_hbm.at[pl.ds(g * 16, 16)])


@functools.partial(
    pl.kernel,
    mesh=_mesh,
    compiler_params=_scp,
    out_type=jax.ShapeDtypeStruct((NPAD,), jnp.float32),
    scratch_types=[
        pltpu.VMEM((SPT, D), jnp.float32),
        pltpu.VMEM((SPT,), jnp.float32),
        pltpu.VMEM((SPT,), jnp.int32),
        pltpu.VMEM((B * 16,), jnp.int32),
        pltpu.VMEM((D,), jnp.float32),
        pltpu.VMEM((L,), jnp.float32),
        pltpu.VMEM((SPT,), jnp.float32),
    ],
)
def _k2c(apc_hbm, t_hbm, wr_hbm, br_hbm, batch_hbm, thr_hbm, w_hbm,
         b0, tb, btb, thrb, wrb, brb, wb):
    cid = lax.axis_index("c")
    sid = lax.axis_index("s")
    wid = sid * NC + cid
    off0 = wid * SPT
    pltpu.sync_copy(apc_hbm.at[pl.ds(off0, SPT)], b0)
    pltpu.sync_copy(t_hbm.at[pl.ds(off0, SPT)], tb)
    pltpu.sync_copy(batch_hbm.at[pl.ds(off0, SPT)], btb)
    pltpu.sync_copy(thr_hbm, thrb)
    pltpu.sync_copy(wr_hbm, wrb)
    pltpu.sync_copy(br_hbm, brb)
    wr8 = [_bf(wrb[pl.ds(cc * L, L)]) for cc in range(D // L)]
    bscal = brb[...][0]
    lane = lax.iota(jnp.int32, L)

    def body(j, _):
        row0 = off0 + j * L
        sval, key = _chunk_keys(b0, wr8, bscal, tb[pl.ds(j * L, L)], lane,
                                row_off=j * L)
        g16 = btb[pl.ds(j * L, L)] * 16
        thr_u = plsc.bitcast(plsc.load_gather(thrb, [g16]), jnp.uint32)
        cut_i = plsc.load_gather(thrb, [g16 + 1])
        idx = row0 + lane
        keep = (key > thr_u) | ((key == thr_u) & (idx <= cut_i))
        wb[pl.ds(j * L, L)] = jnp.where(keep, sval, 0.0)
        return 0

    lax.fori_loop(0, SPT // L, body, 0)
    pltpu.sync_copy(wb, w_hbm.at[pl.ds(off0, SPT)])


def _k3_body(x_ref, br_ref, wr_ref, cnt_ref, out_ref):
    i = pl.program_id(0)
    xb = x_ref[...]
    bt = br_ref[...]
    wv = wr_ref[...]
    gid = lax.broadcasted_iota(jnp.int32, (B, RBS3), 0)
    Wm = jnp.where(bt == gid, wv, 0.0)
    part = jnp.dot(Wm, xb, preferred_element_type=jnp.float32)

    @pl.when(i == 0)
    def _():
        out_ref[...] = part

    @pl.when(i > 0)
    def _():
        out_ref[...] += part

    @pl.when(i == RB - 1)
    def _():
        c = cnt_ref[...][0:1, :]
        kk = (c + 1) // 2
        winv = 1.0 / jnp.maximum(kk, 1).astype(jnp.float32)
        gr = lax.broadcasted_iota(jnp.int32, (B, B), 0)
        gc = lax.broadcasted_iota(jnp.int32, (B, B), 1)
        Dm = jnp.where(gr == gc, jnp.broadcast_to(winv, (B, B)), 0.0)
        out_ref[...] = jnp.dot(Dm, out_ref[...],
                               preferred_element_type=jnp.float32)


_k3 = pl.pallas_call(
    _k3_body,
    grid=(RB,),
    in_specs=[
        pl.BlockSpec((RBS3, D), lambda i: (i, 0)),
        pl.BlockSpec((1, RBS3), lambda i: (0, i)),
        pl.BlockSpec((1, RBS3), lambda i: (0, i)),
        pl.BlockSpec((8, B), lambda i: (0, 0)),
    ],
    out_specs=pl.BlockSpec((B, D), lambda i: (0, 0)),
    out_shape=jax.ShapeDtypeStruct((B, D), jnp.float32),
)


def kernel(x, edge_index, batch, W_rel, b_rel, W_root):
    batch = batch.astype(jnp.int32)
    src = edge_index[0].astype(jnp.int32)
    dst = edge_index[1].astype(jnp.int32)
    t, cnts, strts = _k1(x, batch.reshape(N, 1), W_root.reshape(1, D))
    tpad = jnp.pad(t.reshape(N), (0, NPAD - N))
    srcl, dstl, ecnts = _k2p(src, dst)
    apc = _k2m(x, srcl.reshape(NC * NW, NCH, EC),
               dstl.reshape(NC * NW, NCH, EC), ecnts)
    wr_flat = W_rel.reshape(D)
    br16 = jnp.pad(b_rel, (0, L - 1))
    thrpack = _k2b(apc, tpad, wr_flat, br16, cnts.reshape(-1),
                   strts.reshape(-1))
    w = _k2c(apc, tpad, wr_flat, br16, jnp.pad(batch, (0, NPAD - N)), thrpack)
    xpad = jnp.pad(x, ((0, NPAD - N), (0, 0)))
    bpad = jnp.pad(batch, (0, NPAD - N), constant_values=B)
    return _k3(xpad, bpad.reshape(1, NPAD), w.reshape(1, NPAD), cnts)
